# Initial kernel scaffold; baseline (speedup 1.0000x reference)
import jax
import jax.numpy as jnp
from jax.experimental import pallas as pl


def kernel(x, weight, gamma, beta):
    raise NotImplementedError("write your pallas kernel here")


# speedup vs baseline: 1.0056x; 1.0056x over previous
import functools

import jax
import jax.numpy as jnp
from jax.experimental import pallas as pl
from jax.experimental.pallas import tpu as pltpu


def _round_up(x, m):
    return (x + m - 1) // m * m


def _bnneck_kernel(x_ref, w_ref, gamma_ref, beta_ref, o_ref, *, n_actual):
    y = jax.lax.dot_general(
        x_ref[...], w_ref[...],
        dimension_numbers=(((1,), (1,)), ((), ())),
        preferred_element_type=jnp.float32,
    )
    inv_n = 1.0 / float(n_actual)
    if n_actual == y.shape[0]:
        mean = jnp.sum(y, axis=0, keepdims=True) * inv_n
        diff = y - mean
    else:
        row_ids = jax.lax.broadcasted_iota(jnp.int32, y.shape, 0)
        valid = row_ids < n_actual
        mean = jnp.sum(jnp.where(valid, y, 0.0), axis=0, keepdims=True) * inv_n
        diff = jnp.where(valid, y - mean, 0.0)
    var = jnp.sum(diff * diff, axis=0, keepdims=True) * inv_n
    z = (y - mean) * jax.lax.rsqrt(var + 1e-5)
    z = z * gamma_ref[...] + beta_ref[...]
    z = jnp.where(z >= 0, z, 0.25 * z)
    o_ref[...] = z.astype(o_ref.dtype)


def kernel(x, weight, gamma, beta):
    n, c_in, h, w_sp = x.shape
    assert h == 1 and w_sp == 1
    c_out = weight.shape[0]
    dtype = x.dtype

    cin_p = _round_up(c_in, 128)
    cout_p = _round_up(c_out, 128)
    n_p = _round_up(n, 8)

    x2d = x.reshape(n, c_in)
    w2d = weight.reshape(c_out, c_in)
    if (n_p, cin_p) != (n, c_in):
        x2d = jnp.zeros((n_p, cin_p), dtype).at[:n, :c_in].set(x2d)
    if (cout_p, cin_p) != (c_out, c_in):
        w2d = jnp.zeros((cout_p, cin_p), w2d.dtype).at[:c_out, :c_in].set(w2d)
    gamma2 = gamma.reshape(1, c_out).astype(jnp.float32)
    beta2 = beta.reshape(1, c_out).astype(jnp.float32)
    if cout_p != c_out:
        gamma2 = jnp.zeros((1, cout_p), jnp.float32).at[:, :c_out].set(gamma2)
        beta2 = jnp.zeros((1, cout_p), jnp.float32).at[:, :c_out].set(beta2)

    tile_co = 256 if cout_p % 256 == 0 else 128
    grid = cout_p // tile_co

    out = pl.pallas_call(
        functools.partial(_bnneck_kernel, n_actual=n),
        out_shape=jax.ShapeDtypeStruct((n_p, cout_p), dtype),
        grid=(grid,),
        in_specs=[
            pl.BlockSpec((n_p, cin_p), lambda i: (0, 0)),
            pl.BlockSpec((tile_co, cin_p), lambda i: (i, 0)),
            pl.BlockSpec((1, tile_co), lambda i: (0, i)),
            pl.BlockSpec((1, tile_co), lambda i: (0, i)),
        ],
        out_specs=pl.BlockSpec((n_p, tile_co), lambda i: (0, i)),
        compiler_params=pltpu.CompilerParams(
            dimension_semantics=("parallel",)),
    )(x2d, w2d, gamma2, beta2)
    if (n_p, cout_p) != (n, c_out):
        out = out[:n, :c_out]
    return out


# speedup vs baseline: 1.3263x; 1.3189x over previous
import functools

import jax
import jax.numpy as jnp
from jax.experimental import pallas as pl
from jax.experimental.pallas import tpu as pltpu

_LANES = 128
_N_SUB = 4


def _bnneck_kernel(x_hbm, w_hbm, gamma_ref, beta_ref, o_hbm,
                   x_stage, x_asm, w_stage, w_asm, z_buf,
                   sem_sx, sem_sw, sem_rx, sem_rw, sem_out,
                   *, n, c_in, c_out):
    kj = c_in // _LANES
    co_half = c_out // 2
    sub = co_half // _N_SUB
    i = pl.program_id(0)
    co_base = i * co_half

    x_load = pltpu.make_async_copy(x_hbm, x_stage, sem_sx)

    def w_load(s):
        return pltpu.make_async_copy(
            w_hbm.at[pl.ds(co_base + s * sub, sub)], w_stage.at[s % 2],
            sem_sw.at[s % 2])

    def x_retile(j):
        return pltpu.make_async_copy(
            x_stage.at[:, j, :], x_asm.at[:, pl.ds(j * _LANES, _LANES)],
            sem_rx)

    def w_retile(s, j):
        return pltpu.make_async_copy(
            w_stage.at[s % 2, :, j, :],
            w_asm.at[s % 2, :, pl.ds(j * _LANES, _LANES)], sem_rw.at[s % 2])

    def out_store(s):
        return pltpu.make_async_copy(
            z_buf.at[s % 2], o_hbm.at[:, pl.ds(co_base + s * sub, sub)],
            sem_out.at[s % 2])

    x_load.start()
    w_load(0).start()
    x_load.wait()
    for j in range(kj):
        x_retile(j).start()
    w_load(0).wait()
    for j in range(kj):
        w_retile(0, j).start()
    w_load(1).start()
    for j in range(kj):
        x_retile(j).wait()

    inv_n = 1.0 / float(n)
    for s in range(_N_SUB):
        for j in range(kj):
            w_retile(s, j).wait()
        y = jax.lax.dot_general(
            x_asm[...], w_asm[s % 2],
            dimension_numbers=(((1,), (1,)), ((), ())),
            preferred_element_type=jnp.float32,
        )
        mean = jnp.sum(y, axis=0, keepdims=True) * inv_n
        diff = y - mean
        var = jnp.sum(diff * diff, axis=0, keepdims=True) * inv_n
        z = diff * jax.lax.rsqrt(var + 1e-5)
        cs = pl.ds(s * sub, sub)
        z = z * gamma_ref[:, cs] + beta_ref[:, cs]
        if s >= 2:
            out_store(s - 2).wait()
        z_buf[s % 2, :, :] = jnp.where(z >= 0, z, 0.25 * z)
        out_store(s).start()
        if s + 1 < _N_SUB:
            w_load(s + 1).wait()
            for j in range(kj):
                w_retile(s + 1, j).start()
            if s + 2 < _N_SUB:
                w_load(s + 2).start()
    for s in range(max(0, _N_SUB - 2), _N_SUB):
        out_store(s).wait()


def kernel(x, weight, gamma, beta):
    n, c_in, h, w_sp = x.shape
    assert h == 1 and w_sp == 1
    c_out = weight.shape[0]
    assert n % 8 == 0 and c_in % _LANES == 0
    assert c_out % (2 * _N_SUB * _LANES) == 0
    kj = c_in // _LANES
    co_half = c_out // 2
    sub = co_half // _N_SUB

    x3 = x.reshape(n, kj, _LANES)
    w3 = weight.reshape(c_out, kj, _LANES)
    gamma2 = gamma.reshape(1, c_out).astype(jnp.float32)
    beta2 = beta.reshape(1, c_out).astype(jnp.float32)

    body = functools.partial(_bnneck_kernel, n=n, c_in=c_in, c_out=c_out)
    return pl.pallas_call(
        body,
        out_shape=jax.ShapeDtypeStruct((n, c_out), x.dtype),
        grid=(2,),
        in_specs=[
            pl.BlockSpec(memory_space=pltpu.MemorySpace.HBM),
            pl.BlockSpec(memory_space=pltpu.MemorySpace.HBM),
            pl.BlockSpec((1, co_half), lambda i: (0, i)),
            pl.BlockSpec((1, co_half), lambda i: (0, i)),
        ],
        out_specs=pl.BlockSpec(memory_space=pltpu.MemorySpace.HBM),
        scratch_shapes=[
            pltpu.VMEM((n, kj, _LANES), jnp.float32),
            pltpu.VMEM((n, c_in), jnp.float32),
            pltpu.VMEM((2, sub, kj, _LANES), jnp.float32),
            pltpu.VMEM((2, sub, c_in), jnp.float32),
            pltpu.VMEM((2, n, sub), jnp.float32),
            pltpu.SemaphoreType.DMA,
            pltpu.SemaphoreType.DMA((2,)),
            pltpu.SemaphoreType.DMA,
            pltpu.SemaphoreType.DMA((2,)),
            pltpu.SemaphoreType.DMA((2,)),
        ],
        compiler_params=pltpu.CompilerParams(
            dimension_semantics=("parallel",),
            vmem_limit_bytes=56 * 1024 * 1024,
        ),
    )(x3, w3, gamma2, beta2)


# speedup vs baseline: 2.8065x; 2.1160x over previous
import functools

import jax
import jax.numpy as jnp
from jax.experimental import pallas as pl
from jax.experimental.pallas import tpu as pltpu

_LANES = 128
_N_SUB = 4


def _bnneck_kernel(x_ref, w_ref, gamma_ref, beta_ref, o_ref, x_asm, *, n):
    c_in = x_ref.shape[1] * _LANES

    @pl.when(pl.program_id(1) == 0)
    def _cache_x():
        x_asm[...] = x_ref[...].reshape(n, c_in)

    wk = w_ref[...].reshape(w_ref.shape[0], c_in)
    y = jax.lax.dot_general(
        x_asm[...], wk, dimension_numbers=(((1,), (1,)), ((), ())),
        preferred_element_type=jnp.float32)
    inv_n = 1.0 / float(n)
    mean = jnp.sum(y, axis=0, keepdims=True) * inv_n
    diff = y - mean
    var = jnp.sum(diff * diff, axis=0, keepdims=True) * inv_n
    z = diff * jax.lax.rsqrt(var + 1e-5)
    z = z * gamma_ref[...] + beta_ref[...]
    o_ref[...] = jnp.where(z >= 0, z, 0.25 * z)


def kernel(x, weight, gamma, beta):
    n, c_in, h, w_sp = x.shape
    assert h == 1 and w_sp == 1
    c_out = weight.shape[0]
    assert n % 8 == 0 and c_in % _LANES == 0
    kj = c_in // _LANES
    tile_co = c_out // (2 * _N_SUB)
    assert tile_co % _LANES == 0

    x3 = x.reshape(n, kj, _LANES)
    w3 = weight.reshape(c_out, kj, _LANES)
    gamma2 = gamma.reshape(1, c_out).astype(jnp.float32)
    beta2 = beta.reshape(1, c_out).astype(jnp.float32)

    body = functools.partial(_bnneck_kernel, n=n)
    return pl.pallas_call(
        body,
        out_shape=jax.ShapeDtypeStruct((n, c_out), x.dtype),
        grid=(2, _N_SUB),
        in_specs=[
            pl.BlockSpec((n, kj, _LANES), lambda i, j: (0, 0, 0)),
            pl.BlockSpec((tile_co, kj, _LANES),
                         lambda i, j: (i * _N_SUB + j, 0, 0)),
            pl.BlockSpec((1, tile_co), lambda i, j: (0, i * _N_SUB + j)),
            pl.BlockSpec((1, tile_co), lambda i, j: (0, i * _N_SUB + j)),
        ],
        out_specs=pl.BlockSpec((n, tile_co), lambda i, j: (0, i * _N_SUB + j)),
        scratch_shapes=[pltpu.VMEM((n, c_in), jnp.float32)],
        compiler_params=pltpu.CompilerParams(
            dimension_semantics=("parallel", "arbitrary"),
            vmem_limit_bytes=56 * 1024 * 1024,
        ),
    )(x3, w3, gamma2, beta2)


# speedup vs baseline: 2.8545x; 1.0171x over previous
import functools

import jax
import jax.numpy as jnp
from jax.experimental import pallas as pl
from jax.experimental.pallas import tpu as pltpu

_LANES = 128
_N_SUB = 4


def _bnneck_kernel(x_ref, w_ref, gamma_ref, beta_ref, o_ref, x_asm, *, n):
    c_in = x_ref.shape[1] * _LANES

    @pl.when(pl.program_id(1) == 0)
    def _cache_x():
        x_asm[...] = x_ref[...].astype(jnp.bfloat16).reshape(n, c_in)

    wk = w_ref[...].astype(jnp.bfloat16).reshape(w_ref.shape[0], c_in)
    y = jax.lax.dot_general(
        x_asm[...], wk, dimension_numbers=(((1,), (1,)), ((), ())),
        preferred_element_type=jnp.float32)
    inv_n = 1.0 / float(n)
    mean = jnp.sum(y, axis=0, keepdims=True) * inv_n
    diff = y - mean
    var = jnp.sum(diff * diff, axis=0, keepdims=True) * inv_n
    z = diff * jax.lax.rsqrt(var + 1e-5)
    z = z * gamma_ref[...] + beta_ref[...]
    o_ref[...] = jnp.where(z >= 0, z, 0.25 * z)


def kernel(x, weight, gamma, beta):
    n, c_in, h, w_sp = x.shape
    assert h == 1 and w_sp == 1
    c_out = weight.shape[0]
    assert n % 8 == 0 and c_in % _LANES == 0
    kj = c_in // _LANES
    tile_co = c_out // (2 * _N_SUB)
    assert tile_co % _LANES == 0

    x3 = x.reshape(n, kj, _LANES)
    w3 = weight.reshape(c_out, kj, _LANES)
    gamma2 = gamma.reshape(1, c_out).astype(jnp.float32)
    beta2 = beta.reshape(1, c_out).astype(jnp.float32)

    body = functools.partial(_bnneck_kernel, n=n)
    return pl.pallas_call(
        body,
        out_shape=jax.ShapeDtypeStruct((n, c_out), x.dtype),
        grid=(2, _N_SUB),
        in_specs=[
            pl.BlockSpec((n, kj, _LANES), lambda i, j: (0, 0, 0)),
            pl.BlockSpec((tile_co, kj, _LANES),
                         lambda i, j: (i * _N_SUB + j, 0, 0)),
            pl.BlockSpec((1, tile_co), lambda i, j: (0, i * _N_SUB + j)),
            pl.BlockSpec((1, tile_co), lambda i, j: (0, i * _N_SUB + j)),
        ],
        out_specs=pl.BlockSpec((n, tile_co), lambda i, j: (0, i * _N_SUB + j)),
        scratch_shapes=[pltpu.VMEM((n, c_in), jnp.bfloat16)],
        compiler_params=pltpu.CompilerParams(
            dimension_semantics=("parallel", "arbitrary"),
            vmem_limit_bytes=56 * 1024 * 1024,
        ),
    )(x3, w3, gamma2, beta2)
